# Initial kernel scaffold; baseline (speedup 1.0000x reference)
import jax
import jax.numpy as jnp
from jax.experimental import pallas as pl


def kernel(coordinates, aabb, plane_xy, plane_xz, plane_yz, line_z, line_y, line_x, proj_w, proj_b):
    raise NotImplementedError("write your pallas kernel here")


# speedup vs baseline: 7.8719x; 7.8719x over previous
import functools

import jax
import jax.numpy as jnp
from jax import lax
from jax.experimental import pallas as pl
from jax.experimental.pallas import tpu as pltpu
from jax.experimental.pallas import tpu_sc as plsc

N = 262144
RES = 512
RANK = 48
OUT = 32

NC = 2
NS = 16
LANES = 16
NW = NC * NS
PPW = N // NW
C = 16
NCHUNKS = PPW // C


def _sc_gather_combine(coords_hbm, aabb_hbm, txy_hbm, txz_hbm, tyz_hbm,
                       lz_hbm, ly_hbm, lx_hbm, vm_hbm,
                       coords_v, aabb_v, idx_xy, idx_xz, idx_yz,
                       rows_xy, rows_xz, rows_yz, lz_v, ly_v, lx_v,
                       vm_v, sem):
    wid = lax.axis_index("s") * NC + lax.axis_index("c")
    base = wid * PPW

    pltpu.sync_copy(lz_hbm, lz_v)
    pltpu.sync_copy(ly_hbm, ly_v)
    pltpu.sync_copy(lx_hbm, lx_v)
    pltpu.sync_copy(aabb_hbm, aabb_v)

    iota = lax.iota(jnp.int32, LANES)
    r16 = iota + 16
    r32 = iota + 32
    r48 = iota + 48

    cxc = (aabb_v[0] + aabb_v[3]) * 0.5
    cyc = (aabb_v[1] + aabb_v[4]) * 0.5
    czc = (aabb_v[2] + aabb_v[5]) * 0.5
    hx = jnp.maximum((aabb_v[3] - aabb_v[0]) * 0.5, 1e-6)
    hy = jnp.maximum((aabb_v[4] - aabb_v[1]) * 0.5, 1e-6)
    hz = jnp.maximum((aabb_v[5] - aabb_v[2]) * 0.5, 1e-6)

    def chunk(g, carry):
        p0 = base + g * C
        pltpu.sync_copy(coords_hbm.at[:, pl.ds(p0, C)], coords_v)
        x = (coords_v[0] - cxc) / hx
        y = (coords_v[1] - cyc) / hy
        z = (coords_v[2] - czc) / hz
        linf = jnp.maximum(jnp.maximum(jnp.abs(x), jnp.abs(y)), jnp.abs(z))
        safe = jnp.maximum(linf, 1.0)
        scale = (2.0 - 1.0 / safe) / safe
        big = linf > 1.0
        x = jnp.clip(jnp.where(big, x * scale, x), -1.0, 1.0)
        y = jnp.clip(jnp.where(big, y * scale, y), -1.0, 1.0)
        z = jnp.clip(jnp.where(big, z * scale, z), -1.0, 1.0)

        def grid_coord(c):
            f = (c + 1.0) * 0.5 * (RES - 1)
            i0 = f.astype(jnp.int32)
            w = f - i0.astype(jnp.float32)
            i1 = jnp.minimum(i0 + 1, RES - 1)
            return i0, i1, w

        ix0, ix1, wx = grid_coord(x)
        iy0, iy1, wy = grid_coord(y)
        iz0, iz1, wz = grid_coord(z)

        idx_xy[pl.ds(0, C)] = iy0 * RES + ix0
        idx_xy[pl.ds(16, C)] = iy0 * RES + ix1
        idx_xy[pl.ds(32, C)] = iy1 * RES + ix0
        idx_xy[pl.ds(48, C)] = iy1 * RES + ix1
        idx_xz[pl.ds(0, C)] = iz0 * RES + ix0
        idx_xz[pl.ds(16, C)] = iz0 * RES + ix1
        idx_xz[pl.ds(32, C)] = iz1 * RES + ix0
        idx_xz[pl.ds(48, C)] = iz1 * RES + ix1
        idx_yz[pl.ds(0, C)] = iz0 * RES + iy0
        idx_yz[pl.ds(16, C)] = iz0 * RES + iy1
        idx_yz[pl.ds(32, C)] = iz1 * RES + iy0
        idx_yz[pl.ds(48, C)] = iz1 * RES + iy1

        cp0 = pltpu.async_copy(txy_hbm.at[idx_xy], rows_xy, sem)
        cp1 = pltpu.async_copy(txz_hbm.at[idx_xz], rows_xz, sem)
        cp2 = pltpu.async_copy(tyz_hbm.at[idx_yz], rows_yz, sem)

        wx0 = 1.0 - wx
        wy0 = 1.0 - wy
        wz0 = 1.0 - wz
        w_xy = (wy0 * wx0, wy0 * wx, wy * wx0, wy * wx)
        w_xz = (wz0 * wx0, wz0 * wx, wz * wx0, wz * wx)
        w_yz = (wz0 * wy0, wz0 * wy, wz * wy0, wz * wy)

        cp0.wait()
        cp1.wait()
        cp2.wait()

        for r in range(RANK):
            cr = jnp.full((LANES,), r, jnp.int32)

            def bilin(rows, w4):
                v = plsc.load_gather(rows, [iota, cr]) * w4[0]
                v += plsc.load_gather(rows, [r16, cr]) * w4[1]
                v += plsc.load_gather(rows, [r32, cr]) * w4[2]
                v += plsc.load_gather(rows, [r48, cr]) * w4[3]
                return v

            pxy = bilin(rows_xy, w_xy)
            pxz = bilin(rows_xz, w_xz)
            pyz = bilin(rows_yz, w_yz)

            def lerp(ln, i0, i1, w):
                l0 = plsc.load_gather(ln, [i0, cr])
                l1 = plsc.load_gather(ln, [i1, cr])
                return l0 + w * (l1 - l0)

            vm_r = (pxy * lerp(lz_v, iz0, iz1, wz)
                    + pxz * lerp(ly_v, iy0, iy1, wy)
                    + pyz * lerp(lx_v, ix0, ix1, wx))
            plsc.store_scatter(vm_v, [iota, cr], vm_r)

        pltpu.sync_copy(vm_v, vm_hbm.at[pl.ds(p0, C), :])
        return carry

    lax.fori_loop(0, NCHUNKS, chunk, 0)


def _make_sc_kernel():
    mesh = plsc.VectorSubcoreMesh(core_axis_name="c", subcore_axis_name="s")
    return functools.partial(
        pl.kernel, _sc_gather_combine, mesh=mesh,
        compiler_params=pltpu.CompilerParams(
            needs_layout_passes=False, use_tc_tiling_on_sc=False),
        out_type=jax.ShapeDtypeStruct((N, RANK), jnp.float32),
        scratch_types=[
            pltpu.VMEM((3, C), jnp.float32),
            pltpu.VMEM((6, LANES), jnp.float32),
            pltpu.VMEM((4 * C,), jnp.int32),
            pltpu.VMEM((4 * C,), jnp.int32),
            pltpu.VMEM((4 * C,), jnp.int32),
            pltpu.VMEM((4 * C, RANK), jnp.float32),
            pltpu.VMEM((4 * C, RANK), jnp.float32),
            pltpu.VMEM((4 * C, RANK), jnp.float32),
            pltpu.VMEM((RES, RANK), jnp.float32),
            pltpu.VMEM((RES, RANK), jnp.float32),
            pltpu.VMEM((RES, RANK), jnp.float32),
            pltpu.VMEM((C, RANK), jnp.float32),
            pltpu.SemaphoreType.DMA,
        ],
    )()


def _proj_body(vm_ref, w_ref, b_ref, o_ref):
    o_ref[...] = jnp.dot(vm_ref[...], w_ref[...],
                         preferred_element_type=jnp.float32) + b_ref[...]


def _proj_tc(vm, wt, b2):
    nb = 8192
    return pl.pallas_call(
        _proj_body,
        grid=(N // nb,),
        in_specs=[
            pl.BlockSpec((nb, RANK), lambda i: (i, 0)),
            pl.BlockSpec((RANK, OUT), lambda i: (0, 0)),
            pl.BlockSpec((1, OUT), lambda i: (0, 0)),
        ],
        out_specs=pl.BlockSpec((nb, OUT), lambda i: (i, 0)),
        out_shape=jax.ShapeDtypeStruct((N, OUT), jnp.float32),
    )(vm, wt, b2)


def kernel(coordinates, aabb, plane_xy, plane_xz, plane_yz,
           line_z, line_y, line_x, proj_w, proj_b):
    coords_t = coordinates.T
    aabb_b = jnp.broadcast_to(aabb[:, None], (6, LANES))
    txy = plane_xy.transpose(1, 2, 0).reshape(RES * RES, RANK)
    txz = plane_xz.transpose(1, 2, 0).reshape(RES * RES, RANK)
    tyz = plane_yz.transpose(1, 2, 0).reshape(RES * RES, RANK)
    lz = line_z.T
    ly = line_y.T
    lx = line_x.T

    vm = _make_sc_kernel()(coords_t, aabb_b, txy, txz, tyz, lz, ly, lx)
    return _proj_tc(vm, proj_w.T, proj_b.reshape(1, OUT))


# speedup vs baseline: 7.8783x; 1.0008x over previous
import functools

import jax
import jax.numpy as jnp
from jax import lax
from jax.experimental import pallas as pl
from jax.experimental.pallas import tpu as pltpu
from jax.experimental.pallas import tpu_sc as plsc

N = 262144
RES = 512
RANK = 48
OUT = 32

NC = 2
NS = 16
LANES = 16
NW = NC * NS
PPW = N // NW
C = 16
NCHUNKS = PPW // C
NV = RANK // LANES


def _sc_gather_combine(coords_hbm, aabb_hbm, txy_hbm, txz_hbm, tyz_hbm,
                       lz_hbm, ly_hbm, lx_hbm, vm_hbm,
                       xs_v, ys_v, zs_v, aabb_v,
                       idx_bufs, rows_bufs, lz_v, ly_v, lx_v,
                       vm_bufs,
                       gsem0, gsem1, osem0, osem1):
    wid = lax.axis_index("s") * NC + lax.axis_index("c")
    base = wid * PPW

    pltpu.sync_copy(lz_hbm, lz_v)
    pltpu.sync_copy(ly_hbm, ly_v)
    pltpu.sync_copy(lx_hbm, lx_v)
    pltpu.sync_copy(aabb_hbm, aabb_v)
    pltpu.sync_copy(coords_hbm.at[0, pl.ds(base, PPW)], xs_v)
    pltpu.sync_copy(coords_hbm.at[1, pl.ds(base, PPW)], ys_v)
    pltpu.sync_copy(coords_hbm.at[2, pl.ds(base, PPW)], zs_v)

    cxc = (aabb_v[0] + aabb_v[3]) * 0.5
    cyc = (aabb_v[1] + aabb_v[4]) * 0.5
    czc = (aabb_v[2] + aabb_v[5]) * 0.5
    hx = jnp.maximum((aabb_v[3] - aabb_v[0]) * 0.5, 1e-6)
    hy = jnp.maximum((aabb_v[4] - aabb_v[1]) * 0.5, 1e-6)
    hz = jnp.maximum((aabb_v[5] - aabb_v[2]) * 0.5, 1e-6)

    gsems = (gsem0, gsem1)
    osems = (osem0, osem1)

    def point_coords(g):
        off = (g % NCHUNKS) * C
        x = (xs_v[pl.ds(off, C)] - cxc) / hx
        y = (ys_v[pl.ds(off, C)] - cyc) / hy
        z = (zs_v[pl.ds(off, C)] - czc) / hz
        linf = jnp.maximum(jnp.maximum(jnp.abs(x), jnp.abs(y)), jnp.abs(z))
        safe = jnp.maximum(linf, 1.0)
        scale = (2.0 - 1.0 / safe) / safe
        big = linf > 1.0
        x = jnp.clip(jnp.where(big, x * scale, x), -1.0, 1.0)
        y = jnp.clip(jnp.where(big, y * scale, y), -1.0, 1.0)
        z = jnp.clip(jnp.where(big, z * scale, z), -1.0, 1.0)

        def grid_coord(c):
            f = (c + 1.0) * 0.5 * (RES - 1)
            i0 = f.astype(jnp.int32)
            w = f - i0.astype(jnp.float32)
            i1 = jnp.minimum(i0 + 1, RES - 1)
            return i0, i1, w

        return grid_coord(x), grid_coord(y), grid_coord(z)

    def issue_gathers(g, b):
        (ix0, ix1, _), (iy0, iy1, _), (iz0, iz1, _) = point_coords(g)
        idx = idx_bufs.at[b]
        idx[0, pl.ds(0, C)] = iy0 * RES + ix0
        idx[0, pl.ds(16, C)] = iy0 * RES + ix1
        idx[0, pl.ds(32, C)] = iy1 * RES + ix0
        idx[0, pl.ds(48, C)] = iy1 * RES + ix1
        idx[1, pl.ds(0, C)] = iz0 * RES + ix0
        idx[1, pl.ds(16, C)] = iz0 * RES + ix1
        idx[1, pl.ds(32, C)] = iz1 * RES + ix0
        idx[1, pl.ds(48, C)] = iz1 * RES + ix1
        idx[2, pl.ds(0, C)] = iz0 * RES + iy0
        idx[2, pl.ds(16, C)] = iz0 * RES + iy1
        idx[2, pl.ds(32, C)] = iz1 * RES + iy0
        idx[2, pl.ds(48, C)] = iz1 * RES + iy1
        pltpu.async_copy(txy_hbm.at[idx_bufs.at[b, 0]],
                         rows_bufs.at[b, 0], gsems[b])
        pltpu.async_copy(txz_hbm.at[idx_bufs.at[b, 1]],
                         rows_bufs.at[b, 1], gsems[b])
        pltpu.async_copy(tyz_hbm.at[idx_bufs.at[b, 2]],
                         rows_bufs.at[b, 2], gsems[b])

    def wait_gathers(b):
        for p in range(3):
            pltpu.make_async_copy(txy_hbm.at[idx_bufs.at[b, p]],
                                  rows_bufs.at[b, p], gsems[b]).wait()

    def combine(g, b, h):
        p_prev = base + ((g - 2) % NCHUNKS) * C

        @pl.when(h > 0)
        def _():
            pltpu.make_async_copy(
                vm_bufs.at[b], vm_hbm.at[pl.ds(p_prev, C), :], osems[b]).wait()

        wait_gathers(b)
        (ix0, ix1, wx), (iy0, iy1, wy), (iz0, iz1, wz) = point_coords(g)
        wx0 = 1.0 - wx
        wy0 = 1.0 - wy
        wz0 = 1.0 - wz
        wrows = (wy0 * wx0, wy0 * wx, wy * wx0, wy * wx,
                 wz0 * wx0, wz0 * wx, wz * wx0, wz * wx,
                 wz0 * wy0, wz0 * wy, wz * wy0, wz * wy,
                 wz, wy, wx)
        bz0 = iz0 * RANK
        bz1 = iz1 * RANK
        by0 = iy0 * RANK
        by1 = iy1 * RANK
        bx0 = ix0 * RANK
        bx1 = ix1 * RANK
        rxy = rows_bufs.at[b, 0]
        rxz = rows_bufs.at[b, 1]
        ryz = rows_bufs.at[b, 2]
        vm = vm_bufs.at[b]
        for p in range(C):
            w = [wv[p] for wv in wrows]
            lzb = bz0[p]
            lzb1 = bz1[p]
            lyb = by0[p]
            lyb1 = by1[p]
            lxb = bx0[p]
            lxb1 = bx1[p]
            for v in range(NV):
                o = LANES * v

                def bilin(rows, w4):
                    return (rows[0 * C + p, pl.ds(o, LANES)] * w4[0]
                            + rows[1 * C + p, pl.ds(o, LANES)] * w4[1]
                            + rows[2 * C + p, pl.ds(o, LANES)] * w4[2]
                            + rows[3 * C + p, pl.ds(o, LANES)] * w4[3])

                def lerp(ln, b0, b1, u):
                    l0 = ln[pl.ds(b0 + o, LANES)]
                    l1 = ln[pl.ds(b1 + o, LANES)]
                    return l0 + u * (l1 - l0)

                vm_v = (bilin(rxy, w[0:4]) * lerp(lz_v, lzb, lzb1, w[12])
                        + bilin(rxz, w[4:8]) * lerp(ly_v, lyb, lyb1, w[13])
                        + bilin(ryz, w[8:12]) * lerp(lx_v, lxb, lxb1, w[14]))
                vm[p, pl.ds(o, LANES)] = vm_v

        p0 = base + g * C
        pltpu.async_copy(vm, vm_hbm.at[pl.ds(p0, C), :], osems[b])

    issue_gathers(0, 0)

    def body(h, carry):
        a = 2 * h
        issue_gathers(a + 1, 1)
        combine(a, 0, h)
        issue_gathers(a + 2, 0)
        combine(a + 1, 1, h)
        return carry

    lax.fori_loop(0, NCHUNKS // 2, body, 0)

    for b, g in ((0, NCHUNKS - 2), (1, NCHUNKS - 1)):
        p0 = base + g * C
        pltpu.make_async_copy(
            vm_bufs.at[b], vm_hbm.at[pl.ds(p0, C), :], osems[b]).wait()
    wait_gathers(0)


def _make_sc_kernel():
    mesh = plsc.VectorSubcoreMesh(core_axis_name="c", subcore_axis_name="s")
    return functools.partial(
        pl.kernel, _sc_gather_combine, mesh=mesh,
        compiler_params=pltpu.CompilerParams(
            needs_layout_passes=False, use_tc_tiling_on_sc=False),
        out_type=jax.ShapeDtypeStruct((N, RANK), jnp.float32),
        scratch_types=[
            pltpu.VMEM((PPW,), jnp.float32),
            pltpu.VMEM((PPW,), jnp.float32),
            pltpu.VMEM((PPW,), jnp.float32),
            pltpu.VMEM((6, LANES), jnp.float32),
            pltpu.VMEM((2, 3, 4 * C), jnp.int32),
            pltpu.VMEM((2, 3, 4 * C, RANK), jnp.float32),
            pltpu.VMEM((RES * RANK,), jnp.float32),
            pltpu.VMEM((RES * RANK,), jnp.float32),
            pltpu.VMEM((RES * RANK,), jnp.float32),
            pltpu.VMEM((2, C, RANK), jnp.float32),
            pltpu.SemaphoreType.DMA,
            pltpu.SemaphoreType.DMA,
            pltpu.SemaphoreType.DMA,
            pltpu.SemaphoreType.DMA,
        ],
    )()


def _proj_body(vm_ref, w_ref, b_ref, o_ref):
    o_ref[...] = jnp.dot(vm_ref[...], w_ref[...],
                         preferred_element_type=jnp.float32) + b_ref[...]


def _proj_tc(vm, wt, b2):
    nb = 8192
    return pl.pallas_call(
        _proj_body,
        grid=(N // nb,),
        in_specs=[
            pl.BlockSpec((nb, RANK), lambda i: (i, 0)),
            pl.BlockSpec((RANK, OUT), lambda i: (0, 0)),
            pl.BlockSpec((1, OUT), lambda i: (0, 0)),
        ],
        out_specs=pl.BlockSpec((nb, OUT), lambda i: (i, 0)),
        out_shape=jax.ShapeDtypeStruct((N, OUT), jnp.float32),
    )(vm, wt, b2)


def kernel(coordinates, aabb, plane_xy, plane_xz, plane_yz,
           line_z, line_y, line_x, proj_w, proj_b):
    coords_t = coordinates.T
    aabb_b = jnp.broadcast_to(aabb[:, None], (6, LANES))
    txy = plane_xy.transpose(1, 2, 0).reshape(RES * RES, RANK)
    txz = plane_xz.transpose(1, 2, 0).reshape(RES * RES, RANK)
    tyz = plane_yz.transpose(1, 2, 0).reshape(RES * RES, RANK)
    lz = line_z.T.reshape(RES * RANK)
    ly = line_y.T.reshape(RES * RANK)
    lx = line_x.T.reshape(RES * RANK)

    vm = _make_sc_kernel()(coords_t, aabb_b, txy, txz, tyz, lz, ly, lx)
    return _proj_tc(vm, proj_w.T, proj_b.reshape(1, OUT))
